# baseline (device time: 689973 ns/iter reference)
import jax
import jax.numpy as jnp
from jax import lax
from jax.experimental import pallas as pl
from jax.experimental.pallas import tpu as pltpu

N_DEV = 4
SEG_BYTES_SHAPE = None


def _body(
    x_ref,
    w_ref,
    out_ref,
    xg_ref,
    wv,
    wtmp,
    xc,
    xcb,
    o,
    rows_t,
    load_sem,
    store_sem,
    copy_sem,
    send_sems,
    recv_sems,
):
    m_per, k = x_ref.shape
    n_per = w_ref.shape[1]
    seg = m_per // 2

    my = lax.axis_index("i")
    left = lax.rem(my + N_DEV - 1, N_DEV)
    right = lax.rem(my + 1, N_DEV)
    opp = lax.rem(my + 2, N_DEV)
    u = pl.program_id(0)

    def seg_rdma(src_row, dst_row, sem_i, tgt):
        return pltpu.make_async_remote_copy(
            src_ref=xg_ref.at[pl.ds(src_row, seg), :],
            dst_ref=xg_ref.at[pl.ds(dst_row, seg), :],
            send_sem=send_sems.at[sem_i],
            recv_sem=recv_sems.at[sem_i],
            device_id=(tgt,),
            device_id_type=pl.DeviceIdType.MESH,
        )

    def x_rdma(src_off, dst_row, sem_i, tgt):
        return pltpu.make_async_remote_copy(
            src_ref=x_ref.at[pl.ds(src_off, seg), :],
            dst_ref=xg_ref.at[pl.ds(dst_row, seg), :],
            send_sem=send_sems.at[sem_i],
            recv_sem=recv_sems.at[sem_i],
            device_id=(tgt,),
            device_id_type=pl.DeviceIdType.MESH,
        )

    @pl.when(u == 0)
    def _setup():
        barrier_sem = pltpu.get_barrier_semaphore()
        for nbr in (left, right):
            pl.semaphore_signal(
                barrier_sem,
                inc=1,
                device_id=(nbr,),
                device_id_type=pl.DeviceIdType.MESH,
            )
        pl.semaphore_wait(barrier_sem, 2)

        x_rdma(0, my * m_per, 0, right).start()
        x_rdma(seg, my * m_per + seg, 1, right).start()
        x_rdma(seg, my * m_per + seg, 2, left).start()
        x_rdma(0, my * m_per, 3, left).start()

        loc = pltpu.make_async_copy(
            x_ref, xg_ref.at[pl.ds(my * m_per, m_per), :], copy_sem
        )
        loc.start()

        for j in range(n_per // 512):
            cp = pltpu.make_async_copy(
                w_ref.at[:, pl.ds(j * 512, 512)], wtmp, load_sem
            )
            cp.start()
            cp.wait()
            wv[:, j * 512 : (j + 1) * 512] = wtmp[...].astype(jnp.bfloat16)

        rows_t[0] = my * m_per
        rows_t[1] = my * m_per + seg
        rows_t[2] = left * m_per
        rows_t[3] = right * m_per + seg
        rows_t[4] = left * m_per + seg
        rows_t[5] = right * m_per
        rows_t[6] = opp * m_per
        rows_t[7] = opp * m_per + seg

        loc.wait()

    @pl.when(u == 2)
    def _():
        seg_rdma(left * m_per, left * m_per, 0, right).wait_recv()
        seg_rdma(left * m_per, left * m_per, 4, right).start()

    @pl.when(u == 3)
    def _():
        seg_rdma(right * m_per + seg, right * m_per + seg, 2, left).wait_recv()
        seg_rdma(right * m_per + seg, right * m_per + seg, 5, left).start()

    @pl.when(u == 4)
    def _():
        seg_rdma(left * m_per + seg, left * m_per + seg, 1, right).wait_recv()

    @pl.when(u == 5)
    def _():
        seg_rdma(right * m_per, right * m_per, 3, left).wait_recv()

    @pl.when(u == 6)
    def _():
        seg_rdma(opp * m_per, opp * m_per, 4, right).wait_recv()

    @pl.when(u == 7)
    def _():
        seg_rdma(opp * m_per + seg, opp * m_per + seg, 5, left).wait_recv()

    row = pl.multiple_of(rows_t[u], seg)
    cp = pltpu.make_async_copy(xg_ref.at[pl.ds(row, seg), :], xc, load_sem)
    cp.start()
    cp.wait()
    xcb[...] = xc[...].astype(jnp.bfloat16)
    acc = jnp.dot(xcb[...], wv[...], preferred_element_type=jnp.float32)
    o[...] = acc * jax.nn.sigmoid(acc)
    st = pltpu.make_async_copy(o, out_ref.at[pl.ds(row, seg), :], store_sem)
    st.start()
    st.wait()

    @pl.when(u == 7)
    def _drain():
        for i in range(6):
            seg_rdma(0, 0, i, right).wait_send()


def kernel(x, w_mat):
    m_per, k = x.shape
    _, n_per = w_mat.shape
    m = N_DEV * m_per
    seg = m_per // 2

    out, _xg = pl.pallas_call(
        _body,
        grid=(8,),
        out_shape=[
            jax.ShapeDtypeStruct((m, n_per), jnp.float32),
            jax.ShapeDtypeStruct((m, k), jnp.float32),
        ],
        in_specs=[
            pl.BlockSpec(memory_space=pltpu.MemorySpace.HBM),
            pl.BlockSpec(memory_space=pltpu.MemorySpace.HBM),
        ],
        out_specs=[
            pl.BlockSpec(memory_space=pltpu.MemorySpace.HBM),
            pl.BlockSpec(memory_space=pltpu.MemorySpace.HBM),
        ],
        scratch_shapes=[
            pltpu.VMEM((k, n_per), jnp.bfloat16),
            pltpu.VMEM((k, 512), jnp.float32),
            pltpu.VMEM((seg, k), jnp.float32),
            pltpu.VMEM((seg, k), jnp.bfloat16),
            pltpu.VMEM((seg, n_per), jnp.float32),
            pltpu.SMEM((8,), jnp.int32),
            pltpu.SemaphoreType.DMA,
            pltpu.SemaphoreType.DMA,
            pltpu.SemaphoreType.DMA,
            pltpu.SemaphoreType.DMA((6,)),
            pltpu.SemaphoreType.DMA((6,)),
        ],
        compiler_params=pltpu.CompilerParams(
            dimension_semantics=("arbitrary",),
            collective_id=0,
            vmem_limit_bytes=60 * 1024 * 1024,
        ),
    )(x, w_mat)
    return out


# device time: 415895 ns/iter; 1.6590x vs baseline; 1.6590x over previous
import jax
import jax.numpy as jnp
from jax import lax
from jax.experimental import pallas as pl
from jax.experimental.pallas import tpu as pltpu

N_DEV = 4
SEG_BYTES_SHAPE = None


def _body(
    x_ref,
    w_ref,
    out_ref,
    xg_ref,
    wv,
    wtmp,
    xc,
    xcb,
    o,
    rows_t,
    load_sem,
    store_sem,
    copy_sem,
    send_sems,
    recv_sems,
):
    m_per, k = x_ref.shape
    n_per = w_ref.shape[1]
    seg = m_per // 2

    my = lax.axis_index("i")
    left = lax.rem(my + N_DEV - 1, N_DEV)
    right = lax.rem(my + 1, N_DEV)
    opp = lax.rem(my + 2, N_DEV)
    u = pl.program_id(0)

    def seg_rdma(src_row, dst_row, sem_i, tgt):
        return pltpu.make_async_remote_copy(
            src_ref=xg_ref.at[pl.ds(src_row, seg), :],
            dst_ref=xg_ref.at[pl.ds(dst_row, seg), :],
            send_sem=send_sems.at[sem_i],
            recv_sem=recv_sems.at[sem_i],
            device_id=(tgt,),
            device_id_type=pl.DeviceIdType.MESH,
        )

    def x_rdma(src_off, dst_row, sem_i, tgt):
        return pltpu.make_async_remote_copy(
            src_ref=x_ref.at[pl.ds(src_off, seg), :],
            dst_ref=xg_ref.at[pl.ds(dst_row, seg), :],
            send_sem=send_sems.at[sem_i],
            recv_sem=recv_sems.at[sem_i],
            device_id=(tgt,),
            device_id_type=pl.DeviceIdType.MESH,
        )

    @pl.when(u == 0)
    def _setup():
        barrier_sem = pltpu.get_barrier_semaphore()
        for nbr in (left, right):
            pl.semaphore_signal(
                barrier_sem,
                inc=1,
                device_id=(nbr,),
                device_id_type=pl.DeviceIdType.MESH,
            )
        pl.semaphore_wait(barrier_sem, 2)

        x_rdma(0, my * m_per, 0, right).start()
        x_rdma(seg, my * m_per + seg, 1, right).start()
        x_rdma(seg, my * m_per + seg, 2, left).start()
        x_rdma(0, my * m_per, 3, left).start()

        loc = pltpu.make_async_copy(
            x_ref, xg_ref.at[pl.ds(my * m_per, m_per), :], copy_sem
        )
        loc.start()

        import os as _os

        w_tiles = 0 if _os.environ.get("AG_ONLY") else n_per // 512
        for j in range(w_tiles):
            cp = pltpu.make_async_copy(
                w_ref.at[:, pl.ds(j * 512, 512)], wtmp, load_sem
            )
            cp.start()
            cp.wait()
            wv[:, j * 512 : (j + 1) * 512] = wtmp[...].astype(jnp.bfloat16)

        rows_t[0] = my * m_per
        rows_t[1] = my * m_per + seg
        rows_t[2] = left * m_per
        rows_t[3] = right * m_per + seg
        rows_t[4] = left * m_per + seg
        rows_t[5] = right * m_per
        rows_t[6] = opp * m_per
        rows_t[7] = opp * m_per + seg

        loc.wait()

    @pl.when(u == 2)
    def _():
        seg_rdma(left * m_per, left * m_per, 0, right).wait_recv()
        seg_rdma(left * m_per, left * m_per, 4, right).start()

    @pl.when(u == 3)
    def _():
        seg_rdma(right * m_per + seg, right * m_per + seg, 2, left).wait_recv()
        seg_rdma(right * m_per + seg, right * m_per + seg, 5, left).start()

    @pl.when(u == 4)
    def _():
        seg_rdma(left * m_per + seg, left * m_per + seg, 1, right).wait_recv()

    @pl.when(u == 5)
    def _():
        seg_rdma(right * m_per, right * m_per, 3, left).wait_recv()

    @pl.when(u == 6)
    def _():
        seg_rdma(opp * m_per, opp * m_per, 4, right).wait_recv()

    @pl.when(u == 7)
    def _():
        seg_rdma(opp * m_per + seg, opp * m_per + seg, 5, left).wait_recv()

    import os as _os

    if not _os.environ.get("AG_ONLY"):
        row = pl.multiple_of(rows_t[u], seg)
        cp = pltpu.make_async_copy(
            xg_ref.at[pl.ds(row, seg), :], xc, load_sem
        )
        cp.start()
        cp.wait()
        xcb[...] = xc[...].astype(jnp.bfloat16)
        acc = jnp.dot(xcb[...], wv[...], preferred_element_type=jnp.float32)
        o[...] = acc * jax.nn.sigmoid(acc)
        st = pltpu.make_async_copy(
            o, out_ref.at[pl.ds(row, seg), :], store_sem
        )
        st.start()
        st.wait()

    @pl.when(u == 7)
    def _drain():
        for i in range(6):
            seg_rdma(0, 0, i, right).wait_send()


def _p1_body(x_ref, out_ref, xg_ref, comm, send_sems, recv_sems):
    import os

    variant = os.environ["VARIANT"]
    m_per, k = x_ref.shape
    seg = m_per // 2

    my = lax.axis_index("i")
    left = lax.rem(my + N_DEV - 1, N_DEV)
    right = lax.rem(my + 1, N_DEV)

    barrier_sem = pltpu.get_barrier_semaphore()
    for nbr in (left, right):
        pl.semaphore_signal(
            barrier_sem,
            inc=1,
            device_id=(nbr,),
            device_id_type=pl.DeviceIdType.MESH,
        )
    pl.semaphore_wait(barrier_sem, 2)

    def rdma(src_off, sem_i, tgt, slot):
        if variant == "p1_vv":
            dst = comm.at[slot]
        else:
            dst = xg_ref.at[pl.ds(my * m_per + src_off, seg), :]
        return pltpu.make_async_remote_copy(
            src_ref=x_ref.at[pl.ds(src_off, seg), :],
            dst_ref=dst,
            send_sem=send_sems.at[sem_i],
            recv_sem=recv_sems.at[sem_i],
            device_id=(tgt,),
            device_id_type=pl.DeviceIdType.MESH,
        )

    if variant == "ag12":
        opp = lax.rem(my + 2, N_DEV)
        ds = [
            rdma(0, 0, right, 0),
            rdma(seg, 1, right, 1),
            rdma(seg, 2, left, 2),
            rdma(0, 3, left, 3),
            pltpu.make_async_remote_copy(
                src_ref=x_ref,
                dst_ref=xg_ref.at[pl.ds(my * m_per, m_per), :],
                send_sem=send_sems.at[4],
                recv_sem=recv_sems.at[4],
                device_id=(opp,),
                device_id_type=pl.DeviceIdType.MESH,
            ),
        ]
        for d in ds:
            d.start()
        for d in ds:
            d.wait()
        return

    if variant == "p2_rep":
        ds = [
            rdma(0, 0, right, 0),
            rdma(seg, 1, right, 1),
            rdma(seg, 2, left, 2),
            rdma(0, 3, left, 3),
        ]
        for d in ds:
            d.start()
        for d in ds:
            d.wait()
        es = [
            rdma(0, 4, right, 0),
            rdma(seg, 5, left, 0),
        ]
        for d in es:
            d.start()
        for d in es:
            d.wait()
        return

    if variant == "ag8":
        m = N_DEV * m_per
        left_r = left * m_per
        right_r = right * m_per
        opp = lax.rem(my + 2, N_DEV)

        def fwd(src_row, sem_i, tgt):
            return pltpu.make_async_remote_copy(
                src_ref=xg_ref.at[pl.ds(src_row, seg), :],
                dst_ref=xg_ref.at[pl.ds(src_row, seg), :],
                send_sem=send_sems.at[sem_i],
                recv_sem=recv_sems.at[sem_i],
                device_id=(tgt,),
                device_id_type=pl.DeviceIdType.MESH,
            )

        ds = [
            rdma(0, 0, right, 0),
            rdma(seg, 1, right, 1),
            rdma(seg, 2, left, 2),
            rdma(0, 3, left, 3),
        ]
        for d in ds:
            d.start()
        ds[0].wait_recv()
        ds[2].wait_recv()
        import os as _os

        si_r, si_l = (0, 2) if _os.environ.get("SEM03") else (4, 5)
        ds[0].wait_send()
        rf = fwd(left_r, si_r, right)
        rf.start()
        ds[2].wait_send()
        lf = fwd(right_r + seg, si_l, left)
        lf.start()
        ds[1].wait_recv()
        ds[3].wait_recv()
        fwd(opp * m_per, si_r, right).wait_recv()
        fwd(opp * m_per + seg, si_l, left).wait_recv()
        ds[1].wait_send()
        ds[3].wait_send()
        rf.wait_send()
        lf.wait_send()
        if _os.environ.get("EXITB"):
            import functools

            @functools.partial(
                pl.run_scoped, eb=pltpu.SemaphoreType.REGULAR
            )
            def _(eb):
                for nbr in (left, right):
                    pl.semaphore_signal(
                        eb,
                        inc=1,
                        device_id=(nbr,),
                        device_id_type=pl.DeviceIdType.MESH,
                    )
                pl.semaphore_wait(eb, 2)

        return

    if variant == "p1_24":
        def vflow(rows0, nrows, dst_row, sem_i, tgt):
            return pltpu.make_async_remote_copy(
                src_ref=x_ref.at[pl.ds(rows0, nrows), :],
                dst_ref=xg_ref.at[pl.ds(dst_row, nrows), :],
                send_sem=send_sems.at[sem_i],
                recv_sem=recv_sems.at[sem_i],
                device_id=(tgt,),
                device_id_type=pl.DeviceIdType.MESH,
            )

        vs = [
            vflow(0, 768, my * m_per, 0, right),
            vflow(256, 768, my * m_per + 256, 1, right),
            vflow(0, 768, my * m_per, 2, left),
            vflow(256, 768, my * m_per + 256, 3, left),
        ]
        for d in vs:
            d.start()
        for d in vs:
            d.wait()
        return

    if variant == "ag5":
        m = N_DEV * m_per
        left_r = left * m_per
        right_r = right * m_per
        opp = lax.rem(my + 2, N_DEV)

        def chunk_rdma(sem_i, tgt):
            return pltpu.make_async_remote_copy(
                src_ref=x_ref,
                dst_ref=xg_ref.at[pl.ds(my * m_per, m_per), :],
                send_sem=send_sems.at[sem_i],
                recv_sem=recv_sems.at[sem_i],
                device_id=(tgt,),
                device_id_type=pl.DeviceIdType.MESH,
            )

        def fwd(src_row, sem_i, tgt):
            return pltpu.make_async_remote_copy(
                src_ref=xg_ref.at[pl.ds(src_row, seg), :],
                dst_ref=xg_ref.at[pl.ds(src_row, seg), :],
                send_sem=send_sems.at[sem_i],
                recv_sem=recv_sems.at[sem_i],
                device_id=(tgt,),
                device_id_type=pl.DeviceIdType.MESH,
            )

        cr = chunk_rdma(0, right)
        cl = chunk_rdma(1, left)
        cr.start()
        cl.start()
        cr.wait()
        cl.wait()
        import os as _os

        if _os.environ.get("VSRC"):
            cpa = pltpu.make_async_copy(
                xg_ref.at[pl.ds(left_r, seg), :], comm.at[0], send_sems.at[2]
            )
            cpb = pltpu.make_async_copy(
                xg_ref.at[pl.ds(right_r + seg, seg), :],
                comm.at[1],
                send_sems.at[3],
            )
            cpa.start()
            cpb.start()
            cpa.wait()
            cpb.wait()
            rf = pltpu.make_async_remote_copy(
                src_ref=comm.at[0],
                dst_ref=xg_ref.at[pl.ds(left_r, seg), :],
                send_sem=send_sems.at[4],
                recv_sem=recv_sems.at[4],
                device_id=(right,),
                device_id_type=pl.DeviceIdType.MESH,
            )
            lf = pltpu.make_async_remote_copy(
                src_ref=comm.at[1],
                dst_ref=xg_ref.at[pl.ds(right_r + seg, seg), :],
                send_sem=send_sems.at[5],
                recv_sem=recv_sems.at[5],
                device_id=(left,),
                device_id_type=pl.DeviceIdType.MESH,
            )
        else:
            rf = fwd(left_r, 4, right)
            lf = fwd(right_r + seg, 5, left)
        rf.start()
        lf.start()
        fwd(opp * m_per, 4, right).wait_recv()
        fwd(opp * m_per + seg, 5, left).wait_recv()
        rf.wait_send()
        lf.wait_send()
        return

    if variant == "ag3":
        def chunk_rdma(sem_i, tgt):
            return pltpu.make_async_remote_copy(
                src_ref=x_ref,
                dst_ref=xg_ref.at[pl.ds(my * m_per, m_per), :],
                send_sem=send_sems.at[sem_i],
                recv_sem=recv_sems.at[sem_i],
                device_id=(tgt,),
                device_id_type=pl.DeviceIdType.MESH,
            )

        m = N_DEV * m_per
        left_r = left * m_per
        right_r = right * m_per
        opp = lax.rem(my + 2, N_DEV)

        def fwd(src_row, sem_i, tgt):
            return pltpu.make_async_remote_copy(
                src_ref=xg_ref.at[pl.ds(src_row, seg), :],
                dst_ref=xg_ref.at[pl.ds(src_row, seg), :],
                send_sem=send_sems.at[sem_i],
                recv_sem=recv_sems.at[sem_i],
                device_id=(tgt,),
                device_id_type=pl.DeviceIdType.MESH,
            )

        cr = chunk_rdma(0, right)
        cl = chunk_rdma(1, left)
        cr.start()
        cl.start()
        cr.wait_recv()
        rf = fwd(left_r, 4, right)
        rf.start()
        cl.wait_recv()
        lf = fwd(right_r + seg, 5, left)
        lf.start()
        fwd(opp * m_per, 4, right).wait_recv()
        fwd(opp * m_per + seg, 5, left).wait_recv()
        cr.wait_send()
        cl.wait_send()
        rf.wait_send()
        lf.wait_send()
        return

    ds = [
        rdma(0, 0, right, 0),
        rdma(seg, 1, right, 1),
        rdma(seg, 2, left, 2),
        rdma(0, 3, left, 3),
    ]
    for d in ds:
        d.start()

    if variant == "ag1":
        m = N_DEV * m_per
        my_r = my * m_per
        left_r = left * m_per
        right_r = right * m_per
        opp = lax.rem(my + 2, N_DEV)

        def fwd(src_row, sem_i, tgt):
            return pltpu.make_async_remote_copy(
                src_ref=xg_ref.at[pl.ds(src_row, seg), :],
                dst_ref=xg_ref.at[pl.ds(src_row, seg), :],
                send_sem=send_sems.at[sem_i],
                recv_sem=recv_sems.at[sem_i],
                device_id=(tgt,),
                device_id_type=pl.DeviceIdType.MESH,
            )

        import os as _os

        if _os.environ.get("NODEP"):
            rf = pltpu.make_async_remote_copy(
                src_ref=x_ref.at[pl.ds(0, seg), :],
                dst_ref=xg_ref.at[pl.ds(left_r, seg), :],
                send_sem=send_sems.at[4],
                recv_sem=recv_sems.at[4],
                device_id=(right,),
                device_id_type=pl.DeviceIdType.MESH,
            )
            rf.start()
            lf = pltpu.make_async_remote_copy(
                src_ref=x_ref.at[pl.ds(seg, seg), :],
                dst_ref=xg_ref.at[pl.ds(right_r + seg, seg), :],
                send_sem=send_sems.at[5],
                recv_sem=recv_sems.at[5],
                device_id=(left,),
                device_id_type=pl.DeviceIdType.MESH,
            )
            lf.start()
            ds[0].wait_recv()
            ds[2].wait_recv()
        else:
            ds[0].wait_recv()
            rf = fwd(left_r, 4, right)
            rf.start()
            ds[2].wait_recv()
            lf = fwd(right_r + seg, 5, left)
            lf.start()
        ds[1].wait_recv()
        ds[3].wait_recv()
        fwd(opp * m_per, 4, right).wait_recv()
        fwd(opp * m_per + seg, 5, left).wait_recv()
        for d in ds:
            d.wait_send()
        rf.wait_send()
        lf.wait_send()
    else:
        for d in ds:
            d.wait()


def _run_p1(x, w_mat, variant):
    m_per, k = x.shape
    _, n_per = w_mat.shape
    m = N_DEV * m_per
    seg = m_per // 2
    x_space = (
        pltpu.MemorySpace.HBM
        if variant in ("p1_hbm", "ag1", "ag3", "ag5", "p1_24", "ag8", "p2_rep", "ag12")
        else pltpu.MemorySpace.VMEM
    )
    out, _xg = pl.pallas_call(
        _p1_body,
        out_shape=[
            jax.ShapeDtypeStruct((m, n_per), jnp.float32),
            jax.ShapeDtypeStruct((m, k), jnp.float32),
        ],
        in_specs=[pl.BlockSpec(memory_space=x_space)],
        out_specs=[
            pl.BlockSpec(memory_space=pltpu.MemorySpace.HBM),
            pl.BlockSpec(memory_space=pltpu.MemorySpace.HBM),
        ],
        scratch_shapes=[
            pltpu.VMEM((4, seg, k), jnp.float32),
            pltpu.SemaphoreType.DMA((6,)),
            pltpu.SemaphoreType.DMA((6,)),
        ],
        compiler_params=pltpu.CompilerParams(
            collective_id=0,
            vmem_limit_bytes=60 * 1024 * 1024,
        ),
    )(x)
    return out


def _exchange_body(x_ref, xg_ref, copy_sem, send_sems, recv_sems):
    m_per, k = x_ref.shape
    seg = m_per // 2
    my = lax.axis_index("i")
    left = lax.rem(my + N_DEV - 1, N_DEV)
    right = lax.rem(my + 1, N_DEV)

    barrier_sem = pltpu.get_barrier_semaphore()
    for nbr in (left, right):
        pl.semaphore_signal(
            barrier_sem, inc=1, device_id=(nbr,),
            device_id_type=pl.DeviceIdType.MESH,
        )
    pl.semaphore_wait(barrier_sem, 2)

    def rdma(src_off, sem_i, tgt):
        return pltpu.make_async_remote_copy(
            src_ref=x_ref.at[pl.ds(src_off, seg), :],
            dst_ref=xg_ref.at[pl.ds(my * m_per + src_off, seg), :],
            send_sem=send_sems.at[sem_i],
            recv_sem=recv_sems.at[sem_i],
            device_id=(tgt,),
            device_id_type=pl.DeviceIdType.MESH,
        )

    ds = [
        rdma(0, 0, right),
        rdma(seg, 1, right),
        rdma(seg, 2, left),
        rdma(0, 3, left),
    ]
    for d in ds:
        d.start()
    loc = pltpu.make_async_copy(
        x_ref, xg_ref.at[pl.ds(my * m_per, m_per), :], copy_sem
    )
    loc.start()
    loc.wait()
    for d in ds:
        d.wait()


def _forward_body(xgi_ref, xg_ref, send_sems, recv_sems):
    m = xg_ref.shape[0]
    m_per = m // N_DEV
    seg = m_per // 2
    my = lax.axis_index("i")
    left = lax.rem(my + N_DEV - 1, N_DEV)
    right = lax.rem(my + 1, N_DEV)
    opp = lax.rem(my + 2, N_DEV)

    barrier_sem = pltpu.get_barrier_semaphore()
    for nbr in (left, right):
        pl.semaphore_signal(
            barrier_sem, inc=1, device_id=(nbr,),
            device_id_type=pl.DeviceIdType.MESH,
        )
    pl.semaphore_wait(barrier_sem, 2)

    def fwd(src_row, sem_i, tgt):
        return pltpu.make_async_remote_copy(
            src_ref=xg_ref.at[pl.ds(src_row, seg), :],
            dst_ref=xg_ref.at[pl.ds(src_row, seg), :],
            send_sem=send_sems.at[sem_i],
            recv_sem=recv_sems.at[sem_i],
            device_id=(tgt,),
            device_id_type=pl.DeviceIdType.MESH,
        )

    rf = fwd(left * m_per, 0, right)
    lf = fwd(right * m_per + seg, 1, left)
    rf.start()
    lf.start()
    fwd(opp * m_per, 0, right).wait_recv()
    fwd(opp * m_per + seg, 1, left).wait_recv()
    rf.wait_send()
    lf.wait_send()


def _ag_two_calls(x):
    m_per, k = x.shape
    m = N_DEV * m_per
    xg = pl.pallas_call(
        _exchange_body,
        out_shape=jax.ShapeDtypeStruct((m, k), jnp.float32),
        in_specs=[pl.BlockSpec(memory_space=pltpu.MemorySpace.HBM)],
        out_specs=pl.BlockSpec(memory_space=pltpu.MemorySpace.HBM),
        scratch_shapes=[
            pltpu.SemaphoreType.DMA,
            pltpu.SemaphoreType.DMA((4,)),
            pltpu.SemaphoreType.DMA((4,)),
        ],
        compiler_params=pltpu.CompilerParams(collective_id=0),
    )(x)
    xg = pl.pallas_call(
        _forward_body,
        out_shape=jax.ShapeDtypeStruct((m, k), jnp.float32),
        in_specs=[pl.BlockSpec(memory_space=pltpu.MemorySpace.HBM)],
        out_specs=pl.BlockSpec(memory_space=pltpu.MemorySpace.HBM),
        scratch_shapes=[
            pltpu.SemaphoreType.DMA((2,)),
            pltpu.SemaphoreType.DMA((2,)),
        ],
        input_output_aliases={0: 0},
        compiler_params=pltpu.CompilerParams(collective_id=1),
    )(xg)
    return xg


def _mm_body(x_ref, w_ref, o_ref):
    acc = jnp.dot(
        x_ref[...].astype(jnp.bfloat16),
        w_ref[...].astype(jnp.bfloat16),
        preferred_element_type=jnp.float32,
    )
    o_ref[...] = acc * jax.nn.sigmoid(acc)


def kernel(x, w_mat):
    import os

    variant = os.environ.get("VARIANT", "")
    if variant == "ag13":
        xg = _ag_two_calls(x)
        return xg[:, : w_mat.shape[1]]
    if variant.startswith("p1"):
        return _run_p1(x, w_mat, os.environ["VARIANT"])
    if not variant and not os.environ.get("FUSED"):
        m_per, k = x.shape
        _, n_per = w_mat.shape
        m = N_DEV * m_per
        xg = _ag_two_calls(x)
        bm, bn = 512, 512
        return pl.pallas_call(
            _mm_body,
            out_shape=jax.ShapeDtypeStruct((m, n_per), jnp.float32),
            grid=(m // bm, n_per // bn),
            in_specs=[
                pl.BlockSpec((bm, k), lambda i, j: (i, 0)),
                pl.BlockSpec((k, bn), lambda i, j: (0, j)),
            ],
            out_specs=pl.BlockSpec((bm, bn), lambda i, j: (i, j)),
            compiler_params=pltpu.CompilerParams(
                vmem_limit_bytes=56 * 1024 * 1024
            ),
        )(xg, w_mat)

    m_per, k = x.shape
    _, n_per = w_mat.shape
    m = N_DEV * m_per
    seg = m_per // 2

    out, _xg = pl.pallas_call(
        _body,
        grid=(8,),
        out_shape=[
            jax.ShapeDtypeStruct((m, n_per), jnp.float32),
            jax.ShapeDtypeStruct((m, k), jnp.float32),
        ],
        in_specs=[
            pl.BlockSpec(memory_space=pltpu.MemorySpace.HBM),
            pl.BlockSpec(memory_space=pltpu.MemorySpace.HBM),
        ],
        out_specs=[
            pl.BlockSpec(memory_space=pltpu.MemorySpace.HBM),
            pl.BlockSpec(memory_space=pltpu.MemorySpace.HBM),
        ],
        scratch_shapes=[
            pltpu.VMEM((k, n_per), jnp.bfloat16),
            pltpu.VMEM((k, 512), jnp.float32),
            pltpu.VMEM((seg, k), jnp.float32),
            pltpu.VMEM((seg, k), jnp.bfloat16),
            pltpu.VMEM((seg, n_per), jnp.float32),
            pltpu.SMEM((8,), jnp.int32),
            pltpu.SemaphoreType.DMA,
            pltpu.SemaphoreType.DMA,
            pltpu.SemaphoreType.DMA,
            pltpu.SemaphoreType.DMA((6,)),
            pltpu.SemaphoreType.DMA((6,)),
        ],
        compiler_params=pltpu.CompilerParams(
            dimension_semantics=("arbitrary",),
            collective_id=0,
            vmem_limit_bytes=60 * 1024 * 1024,
        ),
    )(x, w_mat)
    return out
